# baseline (speedup 1.0000x reference)
import functools
import math

import jax
import jax.numpy as jnp
from jax import lax
from jax.experimental import pallas as pl
from jax.experimental.pallas import tpu as pltpu
from jax.experimental.pallas import tpu_sc as plsc

B, N, C, K, H = 4, 2048, 512, 16, 4
DH = C // H
NB = 8
BN = N // NB
NW = 32
PPW = (B * N) // NW


def _h_body(f_ref, Wk1_ref, bk1_ref, h_ref):
    h_ref[...] = (jnp.dot(f_ref[0], Wk1_ref[...].T,
                          preferred_element_type=jnp.float32) + bk1_ref[...])


def _run_h(features, Wk1, bk1):
    return pl.pallas_call(
        _h_body,
        grid=(B, NB),
        in_specs=[
            pl.BlockSpec((1, BN, C), lambda b, nb: (b, nb, 0)),
            pl.BlockSpec((C, C), lambda b, nb: (0, 0)),
            pl.BlockSpec((1, C), lambda b, nb: (0, 0)),
        ],
        out_specs=pl.BlockSpec((BN, C), lambda b, nb: (b * NB + nb, 0)),
        out_shape=jax.ShapeDtypeStruct((B * N, C), jnp.float32),
    )(features, Wk1, bk1)


def _knn_body(xyzp_ref, xyzt_ref, idx_ref):
    b = pl.program_id(0)
    xb = xyzp_ref[0]
    xf = xyzt_ref[0]
    xc0, xc1, xc2 = xb[:, 0:1], xb[:, 1:2], xb[:, 2:3]
    xr0, xr1, xr2 = xf[0:1, :], xf[1:2, :], xf[2:3, :]
    dot = jnp.dot(xb, xf, preferred_element_type=jnp.float32)
    sqb = xc0 * xc0 + xc1 * xc1 + xc2 * xc2
    sqf = xr0 * xr0 + xr1 * xr1 + xr2 * xr2
    d2 = sqb + sqf - 2.0 * dot

    iota = lax.broadcasted_iota(jnp.int32, (BN, N), 1)
    col = lax.broadcasted_iota(jnp.int32, (BN, 128), 1)
    acc = jnp.zeros((BN, 128), jnp.int32)
    work = d2
    for kk in range(K):
        m = jnp.min(work, axis=1, keepdims=True)
        am = jnp.min(jnp.where(work == m, iota, N), axis=1, keepdims=True)
        acc = jnp.where(col == kk, am + b * N, acc)
        work = jnp.where(iota == am, jnp.float32(jnp.inf), work)
    idx_ref[...] = acc


def _run_knn(xyzp, xyzt):
    return pl.pallas_call(
        _knn_body,
        grid=(B, NB),
        in_specs=[
            pl.BlockSpec((1, BN, 128), lambda b, nb: (b, nb, 0)),
            pl.BlockSpec((1, 128, N), lambda b, nb: (b, 0, 0)),
        ],
        out_specs=pl.BlockSpec((BN, 128), lambda b, nb: (b * NB + nb, 0)),
        out_shape=jax.ShapeDtypeStruct((B * N, 128), jnp.int32),
    )(xyzp, xyzt)


def _qkv_body(f_ref, Wi_ref, bi_ref, q_ref, k_ref, v_ref):
    qkv = jnp.dot(f_ref[0], Wi_ref[...].T, preferred_element_type=jnp.float32)
    qkv = qkv + bi_ref[...]
    q_ref[0] = qkv[:, :C]
    k_ref[0] = qkv[:, C:2 * C]
    v_ref[0] = qkv[:, 2 * C:]


def _run_qkv(features, Wi, bi):
    f32 = jnp.float32
    return pl.pallas_call(
        _qkv_body,
        grid=(B, NB),
        in_specs=[
            pl.BlockSpec((1, BN, C), lambda b, nb: (b, nb, 0)),
            pl.BlockSpec((3 * C, C), lambda b, nb: (0, 0)),
            pl.BlockSpec((1, 3 * C), lambda b, nb: (0, 0)),
        ],
        out_specs=[
            pl.BlockSpec((1, BN, C), lambda b, nb: (b, nb, 0)),
            pl.BlockSpec((1, BN, C), lambda b, nb: (b, nb, 0)),
            pl.BlockSpec((1, BN, C), lambda b, nb: (b, nb, 0)),
        ],
        out_shape=[
            jax.ShapeDtypeStruct((B, N, C), f32),
            jax.ShapeDtypeStruct((B, N, C), f32),
            jax.ShapeDtypeStruct((B, N, C), f32),
        ],
    )(features, Wi, bi)


_OB = 8


_NBUF = 4


def _sc_body(h_hbm, idx_hbm, out_hbm, idxv,
             rows0, rows1, rows2, rows3, outs, sem0, sem1, sem2, sem3):
    c = lax.axis_index("c")
    s = lax.axis_index("s")
    wid = s * 2 + c
    base = wid * PPW
    pltpu.sync_copy(idx_hbm.at[pl.ds(base, PPW)], idxv)

    bufs = (rows0, rows1, rows2, rows3)
    sems = (sem0, sem1, sem2, sem3)

    def fire(p, rbuf, sem):
        pltpu.async_copy(h_hbm.at[idxv.at[p, pl.ds(0, K)]], rbuf, sem)

    def wait(p, rbuf, sem):
        pltpu.make_async_copy(h_hbm.at[idxv.at[p, pl.ds(0, K)]], rbuf, sem).wait()

    def reduce_into(rbuf, orow):
        for cc in range(C // 16):
            acc = rbuf[0, pl.ds(cc * 16, 16)]
            for r in range(1, K):
                acc = jnp.maximum(acc, rbuf[r, pl.ds(cc * 16, 16)])
            outs[orow, pl.ds(cc * 16, 16)] = acc

    for j in range(_NBUF - 1):
        fire(j, bufs[j], sems[j])

    def chunk(ob, carry):
        p0 = ob * _OB
        for j in range(_OB):
            b = j % _NBUF
            fb = (j + _NBUF - 1) % _NBUF
            fire(jnp.minimum(p0 + j + _NBUF - 1, PPW - 1), bufs[fb], sems[fb])
            wait(p0 + j, bufs[b], sems[b])
            reduce_into(bufs[b], j)
        pltpu.sync_copy(outs, out_hbm.at[pl.ds(base + p0, _OB)])
        return carry

    lax.fori_loop(0, PPW // _OB, chunk, 0)
    for j in range(_NBUF - 1):
        wait(PPW - 1, bufs[j], sems[j])


def _run_sc_gather_max(h_flat, idx_flat):
    mesh = plsc.VectorSubcoreMesh(core_axis_name="c", subcore_axis_name="s")
    fn = functools.partial(
        pl.kernel,
        mesh=mesh,
        out_type=jax.ShapeDtypeStruct((B * N, C), jnp.float32),
        scratch_types=[
            pltpu.VMEM((PPW, 128), jnp.int32),
            pltpu.VMEM((K, C), jnp.float32),
            pltpu.VMEM((K, C), jnp.float32),
            pltpu.VMEM((K, C), jnp.float32),
            pltpu.VMEM((K, C), jnp.float32),
            pltpu.VMEM((_OB, C), jnp.float32),
            pltpu.SemaphoreType.DMA,
            pltpu.SemaphoreType.DMA,
            pltpu.SemaphoreType.DMA,
            pltpu.SemaphoreType.DMA,
        ],
    )(_sc_body)
    return fn(h_flat, idx_flat)


def _attn_body(q_ref, k_ref, v_ref, Wo_ref, bo_ref, Wm_ref, bm_ref, glob_ref):
    q = q_ref[0]
    kf = k_ref[0]
    vf = v_ref[0]
    scale = 1.0 / math.sqrt(DH)
    parts = []
    for hh in range(H):
        sl = slice(hh * DH, (hh + 1) * DH)
        s = jnp.dot(q[:, sl], kf[:, sl].T,
                    preferred_element_type=jnp.float32) * scale
        s = s - jnp.max(s, axis=1, keepdims=True)
        p = jnp.exp(s)
        p = p / jnp.sum(p, axis=1, keepdims=True)
        parts.append(jnp.dot(p, vf[:, sl], preferred_element_type=jnp.float32))
    ao = jnp.concatenate(parts, axis=1)

    attn_out = jnp.dot(ao, Wo_ref[...].T, preferred_element_type=jnp.float32) + bo_ref[...]
    glob_ref[0] = (jnp.dot(attn_out, Wm_ref[...].T,
                           preferred_element_type=jnp.float32) + bm_ref[...])


def _run_attn(q, k, v, Wo, bo, Wm, bm):
    full = lambda r, c: pl.BlockSpec((r, c), lambda b, nb: (0, 0))
    return pl.pallas_call(
        _attn_body,
        grid=(B, NB),
        in_specs=[
            pl.BlockSpec((1, BN, C), lambda b, nb: (b, nb, 0)),
            pl.BlockSpec((1, N, C), lambda b, nb: (b, 0, 0)),
            pl.BlockSpec((1, N, C), lambda b, nb: (b, 0, 0)),
            full(C, C), full(1, C),
            full(C, C), full(1, C),
        ],
        out_specs=pl.BlockSpec((1, BN, C), lambda b, nb: (b, nb, 0)),
        out_shape=jax.ShapeDtypeStruct((B, N, C), jnp.float32),
    )(q, k, v, Wo, bo, Wm, bm)


def _tail_body(glob_ref, loc_ref, f_ref, Wk2_ref, bk2_ref, Wc_ref, bc_ref,
               g_ref, be_ref, out_ref):
    glob = glob_ref[0]
    loc = jnp.maximum(loc_ref[...].astype(jnp.float32), 0.0)
    loc = jnp.dot(loc, Wk2_ref[...].T, preferred_element_type=jnp.float32) + bk2_ref[...]

    Wc = Wc_ref[...]
    fused = (jnp.dot(glob, Wc[:, :C].T, preferred_element_type=jnp.float32)
             + jnp.dot(loc, Wc[:, C:].T, preferred_element_type=jnp.float32)
             + bc_ref[...])
    fused = jnp.maximum(fused, 0.0)

    x = fused + f_ref[0]
    mu = jnp.mean(x, axis=1, keepdims=True)
    var = jnp.mean((x - mu) * (x - mu), axis=1, keepdims=True)
    out_ref[0] = (x - mu) / jnp.sqrt(var + 1e-5) * g_ref[...] + be_ref[...]


def _run_tail(glob, local_flat, features, Wk2, bk2, Wc, bc, gamma, beta):
    full = lambda r, c: pl.BlockSpec((r, c), lambda b, nb: (0, 0))
    return pl.pallas_call(
        _tail_body,
        grid=(B, NB),
        in_specs=[
            pl.BlockSpec((1, BN, C), lambda b, nb: (b, nb, 0)),
            pl.BlockSpec((BN, C), lambda b, nb: (b * NB + nb, 0)),
            pl.BlockSpec((1, BN, C), lambda b, nb: (b, nb, 0)),
            full(C, C), full(1, C),
            full(C, 2 * C), full(1, C),
            full(1, C), full(1, C),
        ],
        out_specs=pl.BlockSpec((1, BN, C), lambda b, nb: (b, nb, 0)),
        out_shape=jax.ShapeDtypeStruct((B, N, C), jnp.float32),
    )(glob, local_flat, features, Wk2, bk2, Wc, bc, gamma, beta)


def kernel(xyz, features, Wi, bi, Wo, bo, Wm, bm, Wk1, bk1, Wk2, bk2, Wc, bc,
           gamma, beta):
    f32 = jnp.float32
    xyzp = jnp.pad(xyz, ((0, 0), (0, 0), (0, 125))).astype(f32)
    xyzt = jnp.pad(jnp.transpose(xyz, (0, 2, 1)),
                   ((0, 0), (0, 125), (0, 0))).astype(f32)
    r2 = lambda t: t.reshape(1, -1)

    h_flat = _run_h(features, Wk1, r2(bk1))
    idx_flat = _run_knn(xyzp, xyzt)

    local_flat = _run_sc_gather_max(h_flat, idx_flat)

    q, k, v = _run_qkv(features, Wi, r2(bi))
    glob = _run_attn(q, k, v, Wo, r2(bo), Wm, r2(bm))

    return _run_tail(glob, local_flat, features, Wk2, r2(bk2), Wc, r2(bc),
                     r2(gamma), r2(beta))


# speedup vs baseline: 1.2811x; 1.2811x over previous
import functools
import math

import jax
import jax.numpy as jnp
from jax import lax
from jax.experimental import pallas as pl
from jax.experimental.pallas import tpu as pltpu
from jax.experimental.pallas import tpu_sc as plsc

B, N, C, K, H = 4, 2048, 512, 16, 4
DH = C // H
NB = 8
BN = N // NB
NW = 32
PPW = (B * N) // NW


def _h_body(f_ref, Wk1_ref, bk1_ref, h_ref):
    h_ref[...] = (jnp.dot(f_ref[0], Wk1_ref[...].T,
                          preferred_element_type=jnp.float32) + bk1_ref[...])


def _run_h(features, Wk1, bk1):
    return pl.pallas_call(
        _h_body,
        grid=(B, NB),
        in_specs=[
            pl.BlockSpec((1, BN, C), lambda b, nb: (b, nb, 0)),
            pl.BlockSpec((C, C), lambda b, nb: (0, 0)),
            pl.BlockSpec((1, C), lambda b, nb: (0, 0)),
        ],
        out_specs=pl.BlockSpec((BN, C), lambda b, nb: (b * NB + nb, 0)),
        out_shape=jax.ShapeDtypeStruct((B * N, C), jnp.float32),
    )(features, Wk1, bk1)


def _knn_body(xyzp_ref, xyzt_ref, idx_ref):
    b = pl.program_id(0)
    xb = xyzp_ref[0]
    xf = xyzt_ref[0]
    xc0, xc1, xc2 = xb[:, 0:1], xb[:, 1:2], xb[:, 2:3]
    xr0, xr1, xr2 = xf[0:1, :], xf[1:2, :], xf[2:3, :]
    dot = jnp.dot(xb, xf, preferred_element_type=jnp.float32)
    sqb = xc0 * xc0 + xc1 * xc1 + xc2 * xc2
    sqf = xr0 * xr0 + xr1 * xr1 + xr2 * xr2
    d2 = sqb + sqf - 2.0 * dot

    iota = lax.broadcasted_iota(jnp.int32, (BN, N), 1)
    col = lax.broadcasted_iota(jnp.int32, (BN, 128), 1)
    acc = jnp.zeros((BN, 128), jnp.int32)
    work = d2
    for kk in range(K):
        m = jnp.min(work, axis=1, keepdims=True)
        am = jnp.min(jnp.where(work == m, iota, N), axis=1, keepdims=True)
        acc = jnp.where(col == kk, am + b * N, acc)
        work = jnp.where(iota == am, jnp.float32(jnp.inf), work)
    idx_ref[...] = acc


def _run_knn(xyzp, xyzt):
    return pl.pallas_call(
        _knn_body,
        grid=(B, NB),
        in_specs=[
            pl.BlockSpec((1, BN, 128), lambda b, nb: (b, nb, 0)),
            pl.BlockSpec((1, 128, N), lambda b, nb: (b, 0, 0)),
        ],
        out_specs=pl.BlockSpec((BN, 128), lambda b, nb: (b * NB + nb, 0)),
        out_shape=jax.ShapeDtypeStruct((B * N, 128), jnp.int32),
    )(xyzp, xyzt)


def _qkv_body(f_ref, Wi_ref, bi_ref, q_ref, k_ref, v_ref):
    qkv = jnp.dot(f_ref[0], Wi_ref[...].T, preferred_element_type=jnp.float32)
    qkv = qkv + bi_ref[...]
    q_ref[0] = qkv[:, :C]
    k_ref[0] = qkv[:, C:2 * C]
    v_ref[0] = qkv[:, 2 * C:]


def _run_qkv(features, Wi, bi):
    f32 = jnp.float32
    return pl.pallas_call(
        _qkv_body,
        grid=(B, NB),
        in_specs=[
            pl.BlockSpec((1, BN, C), lambda b, nb: (b, nb, 0)),
            pl.BlockSpec((3 * C, C), lambda b, nb: (0, 0)),
            pl.BlockSpec((1, 3 * C), lambda b, nb: (0, 0)),
        ],
        out_specs=[
            pl.BlockSpec((1, BN, C), lambda b, nb: (b, nb, 0)),
            pl.BlockSpec((1, BN, C), lambda b, nb: (b, nb, 0)),
            pl.BlockSpec((1, BN, C), lambda b, nb: (b, nb, 0)),
        ],
        out_shape=[
            jax.ShapeDtypeStruct((B, N, C), f32),
            jax.ShapeDtypeStruct((B, N, C), f32),
            jax.ShapeDtypeStruct((B, N, C), f32),
        ],
    )(features, Wi, bi)


_OB = 64


_NBUF = 4


def _sc_body(h_hbm, idx_hbm, out_hbm, idxv,
             rows0, rows1, rows2, rows3, outs, sem0, sem1, sem2, sem3):
    c = lax.axis_index("c")
    s = lax.axis_index("s")
    wid = s * 2 + c
    base = wid * PPW
    pltpu.sync_copy(idx_hbm.at[pl.ds(base, PPW)], idxv)

    bufs = (rows0, rows1, rows2, rows3)
    sems = (sem0, sem1, sem2, sem3)

    def fire(p, rbuf, sem):
        pltpu.async_copy(h_hbm.at[idxv.at[p, pl.ds(0, K)]], rbuf, sem)

    def wait(p, rbuf, sem):
        pltpu.make_async_copy(h_hbm.at[idxv.at[p, pl.ds(0, K)]], rbuf, sem).wait()

    def reduce_into(rbuf, orow):
        for cc in range(C // 16):
            acc = rbuf[0, pl.ds(cc * 16, 16)]
            for r in range(1, K):
                acc = jnp.maximum(acc, rbuf[r, pl.ds(cc * 16, 16)])
            outs[orow, pl.ds(cc * 16, 16)] = acc

    for j in range(_NBUF):
        fire(j, bufs[j], sems[j])

    def chunk(ob, carry):
        def quad(i, carry2):
            p0 = ob * _OB + _NBUF * i
            for j in range(_NBUF):
                wait(p0 + j, bufs[j], sems[j])
                reduce_into(bufs[j], _NBUF * i + j)
                fire(jnp.minimum(p0 + j + _NBUF, PPW - 1), bufs[j], sems[j])
            return carry2

        lax.fori_loop(0, _OB // _NBUF, quad, 0)
        pltpu.sync_copy(outs, out_hbm.at[pl.ds(base + ob * _OB, _OB)])
        return carry

    lax.fori_loop(0, PPW // _OB, chunk, 0)
    for j in range(_NBUF):
        wait(PPW - 1, bufs[j], sems[j])


def _run_sc_gather_max(h_flat, idx_flat):
    mesh = plsc.VectorSubcoreMesh(core_axis_name="c", subcore_axis_name="s")
    fn = functools.partial(
        pl.kernel,
        mesh=mesh,
        out_type=jax.ShapeDtypeStruct((B * N, C), jnp.float32),
        scratch_types=[
            pltpu.VMEM((PPW, 128), jnp.int32),
            pltpu.VMEM((K, C), jnp.float32),
            pltpu.VMEM((K, C), jnp.float32),
            pltpu.VMEM((K, C), jnp.float32),
            pltpu.VMEM((K, C), jnp.float32),
            pltpu.VMEM((_OB, C), jnp.float32),
            pltpu.SemaphoreType.DMA,
            pltpu.SemaphoreType.DMA,
            pltpu.SemaphoreType.DMA,
            pltpu.SemaphoreType.DMA,
        ],
    )(_sc_body)
    return fn(h_flat, idx_flat)


def _attn_body(q_ref, k_ref, v_ref, Wo_ref, bo_ref, Wm_ref, bm_ref, glob_ref):
    q = q_ref[0]
    kf = k_ref[0]
    vf = v_ref[0]
    scale = 1.0 / math.sqrt(DH)
    parts = []
    for hh in range(H):
        sl = slice(hh * DH, (hh + 1) * DH)
        s = jnp.dot(q[:, sl], kf[:, sl].T,
                    preferred_element_type=jnp.float32) * scale
        s = s - jnp.max(s, axis=1, keepdims=True)
        p = jnp.exp(s)
        p = p / jnp.sum(p, axis=1, keepdims=True)
        parts.append(jnp.dot(p, vf[:, sl], preferred_element_type=jnp.float32))
    ao = jnp.concatenate(parts, axis=1)

    attn_out = jnp.dot(ao, Wo_ref[...].T, preferred_element_type=jnp.float32) + bo_ref[...]
    glob_ref[0] = (jnp.dot(attn_out, Wm_ref[...].T,
                           preferred_element_type=jnp.float32) + bm_ref[...])


def _run_attn(q, k, v, Wo, bo, Wm, bm):
    full = lambda r, c: pl.BlockSpec((r, c), lambda b, nb: (0, 0))
    return pl.pallas_call(
        _attn_body,
        grid=(B, NB),
        in_specs=[
            pl.BlockSpec((1, BN, C), lambda b, nb: (b, nb, 0)),
            pl.BlockSpec((1, N, C), lambda b, nb: (b, 0, 0)),
            pl.BlockSpec((1, N, C), lambda b, nb: (b, 0, 0)),
            full(C, C), full(1, C),
            full(C, C), full(1, C),
        ],
        out_specs=pl.BlockSpec((1, BN, C), lambda b, nb: (b, nb, 0)),
        out_shape=jax.ShapeDtypeStruct((B, N, C), jnp.float32),
    )(q, k, v, Wo, bo, Wm, bm)


def _tail_body(glob_ref, loc_ref, f_ref, Wk2_ref, bk2_ref, Wc_ref, bc_ref,
               g_ref, be_ref, out_ref):
    glob = glob_ref[0]
    loc = jnp.maximum(loc_ref[...].astype(jnp.float32), 0.0)
    loc = jnp.dot(loc, Wk2_ref[...].T, preferred_element_type=jnp.float32) + bk2_ref[...]

    Wc = Wc_ref[...]
    fused = (jnp.dot(glob, Wc[:, :C].T, preferred_element_type=jnp.float32)
             + jnp.dot(loc, Wc[:, C:].T, preferred_element_type=jnp.float32)
             + bc_ref[...])
    fused = jnp.maximum(fused, 0.0)

    x = fused + f_ref[0]
    mu = jnp.mean(x, axis=1, keepdims=True)
    var = jnp.mean((x - mu) * (x - mu), axis=1, keepdims=True)
    out_ref[0] = (x - mu) / jnp.sqrt(var + 1e-5) * g_ref[...] + be_ref[...]


def _run_tail(glob, local_flat, features, Wk2, bk2, Wc, bc, gamma, beta):
    full = lambda r, c: pl.BlockSpec((r, c), lambda b, nb: (0, 0))
    return pl.pallas_call(
        _tail_body,
        grid=(B, NB),
        in_specs=[
            pl.BlockSpec((1, BN, C), lambda b, nb: (b, nb, 0)),
            pl.BlockSpec((BN, C), lambda b, nb: (b * NB + nb, 0)),
            pl.BlockSpec((1, BN, C), lambda b, nb: (b, nb, 0)),
            full(C, C), full(1, C),
            full(C, 2 * C), full(1, C),
            full(1, C), full(1, C),
        ],
        out_specs=pl.BlockSpec((1, BN, C), lambda b, nb: (b, nb, 0)),
        out_shape=jax.ShapeDtypeStruct((B, N, C), jnp.float32),
    )(glob, local_flat, features, Wk2, bk2, Wc, bc, gamma, beta)


def kernel(xyz, features, Wi, bi, Wo, bo, Wm, bm, Wk1, bk1, Wk2, bk2, Wc, bc,
           gamma, beta):
    f32 = jnp.float32
    xyzp = jnp.pad(xyz, ((0, 0), (0, 0), (0, 125))).astype(f32)
    xyzt = jnp.pad(jnp.transpose(xyz, (0, 2, 1)),
                   ((0, 0), (0, 125), (0, 0))).astype(f32)
    r2 = lambda t: t.reshape(1, -1)

    h_flat = _run_h(features, Wk1, r2(bk1))
    idx_flat = _run_knn(xyzp, xyzt)

    local_flat = _run_sc_gather_max(h_flat, idx_flat)

    q, k, v = _run_qkv(features, Wi, r2(bi))
    glob = _run_attn(q, k, v, Wo, r2(bo), Wm, r2(bm))

    return _run_tail(glob, local_flat, features, Wk2, r2(bk2), Wc, r2(bc),
                     r2(gamma), r2(beta))


# speedup vs baseline: 1.3973x; 1.0907x over previous
import functools
import math

import jax
import jax.numpy as jnp
from jax import lax
from jax.experimental import pallas as pl
from jax.experimental.pallas import tpu as pltpu
from jax.experimental.pallas import tpu_sc as plsc

B, N, C, K, H = 4, 2048, 512, 16, 4
DH = C // H
NB = 8
BN = N // NB
NW = 32
PPW = (B * N) // NW


def _h_body(f_ref, Wk1_ref, bk1_ref, h_ref):
    h_ref[...] = (jnp.dot(f_ref[0], Wk1_ref[...].T,
                          preferred_element_type=jnp.float32) + bk1_ref[...])


def _run_h(features, Wk1, bk1):
    return pl.pallas_call(
        _h_body,
        grid=(B, NB),
        in_specs=[
            pl.BlockSpec((1, BN, C), lambda b, nb: (b, nb, 0)),
            pl.BlockSpec((C, C), lambda b, nb: (0, 0)),
            pl.BlockSpec((1, C), lambda b, nb: (0, 0)),
        ],
        out_specs=pl.BlockSpec((BN, C), lambda b, nb: (b * NB + nb, 0)),
        out_shape=jax.ShapeDtypeStruct((B * N, C), jnp.float32),
    )(features, Wk1, bk1)


def _knn_body(xyzp_ref, xyzt_ref, idx_ref):
    b = pl.program_id(0)
    xb = xyzp_ref[0]
    xf = xyzt_ref[0]
    xc0, xc1, xc2 = xb[:, 0:1], xb[:, 1:2], xb[:, 2:3]
    xr0, xr1, xr2 = xf[0:1, :], xf[1:2, :], xf[2:3, :]
    dot = jnp.dot(xb, xf, preferred_element_type=jnp.float32)
    sqb = xc0 * xc0 + xc1 * xc1 + xc2 * xc2
    sqf = xr0 * xr0 + xr1 * xr1 + xr2 * xr2
    d2 = sqb + sqf - 2.0 * dot

    iota = lax.broadcasted_iota(jnp.int32, (BN, N), 1)
    col = lax.broadcasted_iota(jnp.int32, (BN, 128), 1)
    acc = jnp.zeros((BN, 128), jnp.int32)
    work = d2
    for kk in range(K):
        m = jnp.min(work, axis=1, keepdims=True)
        am = jnp.min(jnp.where(work == m, iota, N), axis=1, keepdims=True)
        acc = jnp.where(col == kk, am + b * N, acc)
        work = jnp.where(iota == am, jnp.float32(jnp.inf), work)
    idx_ref[...] = acc


def _run_knn(xyzp, xyzt):
    return pl.pallas_call(
        _knn_body,
        grid=(B, NB),
        in_specs=[
            pl.BlockSpec((1, BN, 128), lambda b, nb: (b, nb, 0)),
            pl.BlockSpec((1, 128, N), lambda b, nb: (b, 0, 0)),
        ],
        out_specs=pl.BlockSpec((BN, 128), lambda b, nb: (b * NB + nb, 0)),
        out_shape=jax.ShapeDtypeStruct((B * N, 128), jnp.int32),
    )(xyzp, xyzt)


def _qkv_body(f_ref, Wi_ref, bi_ref, q_ref, k_ref, v_ref):
    qkv = jnp.dot(f_ref[0], Wi_ref[...].T, preferred_element_type=jnp.float32)
    qkv = qkv + bi_ref[...]
    q_ref[0] = qkv[:, :C]
    k_ref[0] = qkv[:, C:2 * C]
    v_ref[0] = qkv[:, 2 * C:]


def _run_qkv(features, Wi, bi):
    f32 = jnp.float32
    return pl.pallas_call(
        _qkv_body,
        grid=(B, NB),
        in_specs=[
            pl.BlockSpec((1, BN, C), lambda b, nb: (b, nb, 0)),
            pl.BlockSpec((3 * C, C), lambda b, nb: (0, 0)),
            pl.BlockSpec((1, 3 * C), lambda b, nb: (0, 0)),
        ],
        out_specs=[
            pl.BlockSpec((1, BN, C), lambda b, nb: (b, nb, 0)),
            pl.BlockSpec((1, BN, C), lambda b, nb: (b, nb, 0)),
            pl.BlockSpec((1, BN, C), lambda b, nb: (b, nb, 0)),
        ],
        out_shape=[
            jax.ShapeDtypeStruct((B, N, C), f32),
            jax.ShapeDtypeStruct((B, N, C), f32),
            jax.ShapeDtypeStruct((B, N, C), f32),
        ],
    )(features, Wi, bi)


_OB = 64


def _sc_body(h_hbm, idx_hbm, out_hbm, idxv, rows0, rows1, outs, sem0, sem1):
    c = lax.axis_index("c")
    s = lax.axis_index("s")
    wid = s * 2 + c
    base = wid * PPW
    pltpu.sync_copy(idx_hbm.at[pl.ds(base, PPW)], idxv)

    def fire(p, rbuf, sem):
        pltpu.async_copy(h_hbm.at[idxv.at[p, pl.ds(0, K)]], rbuf, sem)

    def wait(p, rbuf, sem):
        pltpu.make_async_copy(h_hbm.at[idxv.at[p, pl.ds(0, K)]], rbuf, sem).wait()

    def reduce_into(rbuf, orow):
        for cc in range(C // 16):
            acc = rbuf[0, pl.ds(cc * 16, 16)]
            for r in range(1, K):
                acc = jnp.maximum(acc, rbuf[r, pl.ds(cc * 16, 16)])
            outs[orow, pl.ds(cc * 16, 16)] = acc

    fire(0, rows0, sem0)
    for ob in range(PPW // _OB):
        def pair(i, carry):
            p0 = ob * _OB + 2 * i
            p1 = p0 + 1
            pn = jnp.minimum(p1 + 1, PPW - 1)
            fire(p1, rows1, sem1)
            wait(p0, rows0, sem0)
            reduce_into(rows0, 2 * i)
            fire(pn, rows0, sem0)
            wait(p1, rows1, sem1)
            reduce_into(rows1, 2 * i + 1)
            return carry

        lax.fori_loop(0, _OB // 2, pair, 0)
        pltpu.sync_copy(outs, out_hbm.at[pl.ds(base + ob * _OB, _OB)])
    wait(PPW - 1, rows0, sem0)


def _run_sc_gather_max(h_flat, idx_flat):
    mesh = plsc.VectorSubcoreMesh(core_axis_name="c", subcore_axis_name="s")
    fn = functools.partial(
        pl.kernel,
        mesh=mesh,
        out_type=jax.ShapeDtypeStruct((B * N, C), jnp.float32),
        scratch_types=[
            pltpu.VMEM((PPW, 128), jnp.int32),
            pltpu.VMEM((K, C), jnp.float32),
            pltpu.VMEM((K, C), jnp.float32),
            pltpu.VMEM((_OB, C), jnp.float32),
            pltpu.SemaphoreType.DMA,
            pltpu.SemaphoreType.DMA,
        ],
    )(_sc_body)
    return fn(h_flat, idx_flat)


def _attn_body(q_ref, k_ref, v_ref, Wo_ref, bo_ref, Wm_ref, bm_ref, glob_ref):
    bf16 = jnp.bfloat16
    q = q_ref[0].astype(bf16)
    kf = k_ref[0].astype(bf16)
    vf = v_ref[0].astype(bf16)
    scale = 1.0 / math.sqrt(DH)
    parts = []
    for hh in range(H):
        sl = slice(hh * DH, (hh + 1) * DH)
        s = jnp.dot(q[:, sl], kf[:, sl].T,
                    preferred_element_type=jnp.float32) * scale
        s = s - jnp.max(s, axis=1, keepdims=True)
        p = jnp.exp(s)
        p = (p / jnp.sum(p, axis=1, keepdims=True)).astype(bf16)
        parts.append(jnp.dot(p, vf[:, sl], preferred_element_type=jnp.float32))
    ao = jnp.concatenate(parts, axis=1)

    attn_out = jnp.dot(ao, Wo_ref[...].T, preferred_element_type=jnp.float32) + bo_ref[...]
    glob_ref[0] = (jnp.dot(attn_out, Wm_ref[...].T,
                           preferred_element_type=jnp.float32) + bm_ref[...])


def _run_attn(q, k, v, Wo, bo, Wm, bm):
    full = lambda r, c: pl.BlockSpec((r, c), lambda b, nb: (0, 0))
    return pl.pallas_call(
        _attn_body,
        grid=(B, NB),
        in_specs=[
            pl.BlockSpec((1, BN, C), lambda b, nb: (b, nb, 0)),
            pl.BlockSpec((1, N, C), lambda b, nb: (b, 0, 0)),
            pl.BlockSpec((1, N, C), lambda b, nb: (b, 0, 0)),
            full(C, C), full(1, C),
            full(C, C), full(1, C),
        ],
        out_specs=pl.BlockSpec((1, BN, C), lambda b, nb: (b, nb, 0)),
        out_shape=jax.ShapeDtypeStruct((B, N, C), jnp.float32),
    )(q, k, v, Wo, bo, Wm, bm)


def _tail_body(glob_ref, loc_ref, f_ref, Wk2_ref, bk2_ref, Wc_ref, bc_ref,
               g_ref, be_ref, out_ref):
    glob = glob_ref[0]
    loc = jnp.maximum(loc_ref[...].astype(jnp.float32), 0.0)
    loc = jnp.dot(loc, Wk2_ref[...].T, preferred_element_type=jnp.float32) + bk2_ref[...]

    Wc = Wc_ref[...]
    fused = (jnp.dot(glob, Wc[:, :C].T, preferred_element_type=jnp.float32)
             + jnp.dot(loc, Wc[:, C:].T, preferred_element_type=jnp.float32)
             + bc_ref[...])
    fused = jnp.maximum(fused, 0.0)

    x = fused + f_ref[0]
    mu = jnp.mean(x, axis=1, keepdims=True)
    var = jnp.mean((x - mu) * (x - mu), axis=1, keepdims=True)
    out_ref[0] = (x - mu) / jnp.sqrt(var + 1e-5) * g_ref[...] + be_ref[...]


def _run_tail(glob, local_flat, features, Wk2, bk2, Wc, bc, gamma, beta):
    full = lambda r, c: pl.BlockSpec((r, c), lambda b, nb: (0, 0))
    return pl.pallas_call(
        _tail_body,
        grid=(B, NB),
        in_specs=[
            pl.BlockSpec((1, BN, C), lambda b, nb: (b, nb, 0)),
            pl.BlockSpec((BN, C), lambda b, nb: (b * NB + nb, 0)),
            pl.BlockSpec((1, BN, C), lambda b, nb: (b, nb, 0)),
            full(C, C), full(1, C),
            full(C, 2 * C), full(1, C),
            full(1, C), full(1, C),
        ],
        out_specs=pl.BlockSpec((1, BN, C), lambda b, nb: (b, nb, 0)),
        out_shape=jax.ShapeDtypeStruct((B, N, C), jnp.float32),
    )(glob, local_flat, features, Wk2, bk2, Wc, bc, gamma, beta)


def kernel(xyz, features, Wi, bi, Wo, bo, Wm, bm, Wk1, bk1, Wk2, bk2, Wc, bc,
           gamma, beta):
    f32 = jnp.float32
    xyzp = jnp.pad(xyz, ((0, 0), (0, 0), (0, 125))).astype(f32)
    xyzt = jnp.pad(jnp.transpose(xyz, (0, 2, 1)),
                   ((0, 0), (0, 125), (0, 0))).astype(f32)
    r2 = lambda t: t.reshape(1, -1)

    h_flat = _run_h(features, Wk1, r2(bk1))
    idx_flat = _run_knn(xyzp, xyzt)

    local_flat = _run_sc_gather_max(h_flat, idx_flat)

    q, k, v = _run_qkv(features, Wi, r2(bi))
    glob = _run_attn(q, k, v, Wo, r2(bo), Wm, r2(bm))

    return _run_tail(glob, local_flat, features, Wk2, r2(bk2), Wc, r2(bc),
                     r2(gamma), r2(beta))


# speedup vs baseline: 1.4323x; 1.0251x over previous
import functools
import math

import jax
import jax.numpy as jnp
from jax import lax
from jax.experimental import pallas as pl
from jax.experimental.pallas import tpu as pltpu
from jax.experimental.pallas import tpu_sc as plsc

B, N, C, K, H = 4, 2048, 512, 16, 4
DH = C // H
NB = 8
BN = N // NB
NW = 32
PPW = (B * N) // NW


def _h_body(f_ref, Wk1_ref, bk1_ref, h_ref):
    h_ref[...] = (jnp.dot(f_ref[0], Wk1_ref[...].T,
                          preferred_element_type=jnp.float32) + bk1_ref[...])


def _run_h(features, Wk1, bk1):
    return pl.pallas_call(
        _h_body,
        grid=(B, NB),
        in_specs=[
            pl.BlockSpec((1, BN, C), lambda b, nb: (b, nb, 0)),
            pl.BlockSpec((C, C), lambda b, nb: (0, 0)),
            pl.BlockSpec((1, C), lambda b, nb: (0, 0)),
        ],
        out_specs=pl.BlockSpec((BN, C), lambda b, nb: (b * NB + nb, 0)),
        out_shape=jax.ShapeDtypeStruct((B * N, C), jnp.float32),
    )(features, Wk1, bk1)


def _knn_body(b0, xyzp_ref, xyzt_ref, idx_ref):
    b = pl.program_id(0) + b0
    xb = xyzp_ref[0]
    xf = xyzt_ref[0]
    xc0, xc1, xc2 = xb[:, 0:1], xb[:, 1:2], xb[:, 2:3]
    xr0, xr1, xr2 = xf[0:1, :], xf[1:2, :], xf[2:3, :]
    dot = jnp.dot(xb, xf, preferred_element_type=jnp.float32)
    sqb = xc0 * xc0 + xc1 * xc1 + xc2 * xc2
    sqf = xr0 * xr0 + xr1 * xr1 + xr2 * xr2
    d2 = sqb + sqf - 2.0 * dot

    iota = lax.broadcasted_iota(jnp.int32, (BN, N), 1)
    col = lax.broadcasted_iota(jnp.int32, (BN, 128), 1)
    acc = jnp.zeros((BN, 128), jnp.int32)
    work = d2
    for kk in range(K):
        m = jnp.min(work, axis=1, keepdims=True)
        am = jnp.min(jnp.where(work == m, iota, N), axis=1, keepdims=True)
        acc = jnp.where(col == kk, am + b * N, acc)
        work = jnp.where(iota == am, jnp.float32(jnp.inf), work)
    idx_ref[...] = acc


def _run_knn(xyzp, xyzt, b0, nbat):
    return pl.pallas_call(
        functools.partial(_knn_body, b0),
        grid=(nbat, NB),
        in_specs=[
            pl.BlockSpec((1, BN, 128), lambda b, nb: (b0 + b, nb, 0)),
            pl.BlockSpec((1, 128, N), lambda b, nb: (b0 + b, 0, 0)),
        ],
        out_specs=pl.BlockSpec((BN, 128), lambda b, nb: (b * NB + nb, 0)),
        out_shape=jax.ShapeDtypeStruct((nbat * N, 128), jnp.int32),
    )(xyzp, xyzt)


def _qkv_body(f_ref, Wi_ref, bi_ref, q_ref, k_ref, v_ref):
    qkv = jnp.dot(f_ref[0], Wi_ref[...].T, preferred_element_type=jnp.float32)
    qkv = qkv + bi_ref[...]
    q_ref[0] = qkv[:, :C]
    k_ref[0] = qkv[:, C:2 * C]
    v_ref[0] = qkv[:, 2 * C:]


def _run_qkv(features, Wi, bi):
    f32 = jnp.float32
    return pl.pallas_call(
        _qkv_body,
        grid=(B, NB),
        in_specs=[
            pl.BlockSpec((1, BN, C), lambda b, nb: (b, nb, 0)),
            pl.BlockSpec((3 * C, C), lambda b, nb: (0, 0)),
            pl.BlockSpec((1, 3 * C), lambda b, nb: (0, 0)),
        ],
        out_specs=[
            pl.BlockSpec((1, BN, C), lambda b, nb: (b, nb, 0)),
            pl.BlockSpec((1, BN, C), lambda b, nb: (b, nb, 0)),
            pl.BlockSpec((1, BN, C), lambda b, nb: (b, nb, 0)),
        ],
        out_shape=[
            jax.ShapeDtypeStruct((B, N, C), f32),
            jax.ShapeDtypeStruct((B, N, C), f32),
            jax.ShapeDtypeStruct((B, N, C), f32),
        ],
    )(features, Wi, bi)


_OB = 64


def _sc_body(ppw, h_hbm, idx_hbm, out_hbm, idxv, rows0, rows1, outs,
             sem0, sem1):
    c = lax.axis_index("c")
    s = lax.axis_index("s")
    wid = s * 2 + c
    base = wid * ppw
    pltpu.sync_copy(idx_hbm.at[pl.ds(base, ppw)], idxv)

    def fire(p, rbuf, sem):
        pltpu.async_copy(h_hbm.at[idxv.at[p, pl.ds(0, K)]], rbuf, sem)

    def wait(p, rbuf, sem):
        pltpu.make_async_copy(h_hbm.at[idxv.at[p, pl.ds(0, K)]], rbuf, sem).wait()

    def reduce_into(rbuf, orow):
        for cc in range(C // 16):
            acc = rbuf[0, pl.ds(cc * 16, 16)]
            for r in range(1, K):
                acc = jnp.maximum(acc, rbuf[r, pl.ds(cc * 16, 16)])
            outs[orow, pl.ds(cc * 16, 16)] = acc

    fire(0, rows0, sem0)
    for ob in range(ppw // _OB):
        def pair(i, carry):
            p0 = ob * _OB + 2 * i
            p1 = p0 + 1
            pn = jnp.minimum(p1 + 1, ppw - 1)
            fire(p1, rows1, sem1)
            wait(p0, rows0, sem0)
            reduce_into(rows0, 2 * i)
            fire(pn, rows0, sem0)
            wait(p1, rows1, sem1)
            reduce_into(rows1, 2 * i + 1)
            return carry

        lax.fori_loop(0, _OB // 2, pair, 0)
        pltpu.sync_copy(outs, out_hbm.at[pl.ds(base + ob * _OB, _OB)])
    wait(ppw - 1, rows0, sem0)


def _run_sc_gather_max(h_flat, idx_flat):
    tp = idx_flat.shape[0]
    ppw = tp // NW
    mesh = plsc.VectorSubcoreMesh(core_axis_name="c", subcore_axis_name="s")
    fn = functools.partial(
        pl.kernel,
        mesh=mesh,
        out_type=jax.ShapeDtypeStruct((tp, C), jnp.float32),
        scratch_types=[
            pltpu.VMEM((ppw, 128), jnp.int32),
            pltpu.VMEM((K, C), jnp.float32),
            pltpu.VMEM((K, C), jnp.float32),
            pltpu.VMEM((_OB, C), jnp.float32),
            pltpu.SemaphoreType.DMA,
            pltpu.SemaphoreType.DMA,
        ],
    )(functools.partial(_sc_body, ppw))
    return fn(h_flat, idx_flat)


def _attn_body(q_ref, k_ref, v_ref, Wo_ref, bo_ref, Wm_ref, bm_ref, glob_ref):
    bf16 = jnp.bfloat16
    q = q_ref[0].astype(bf16)
    kf = k_ref[0].astype(bf16)
    vf = v_ref[0].astype(bf16)
    scale = 1.0 / math.sqrt(DH)
    parts = []
    for hh in range(H):
        sl = slice(hh * DH, (hh + 1) * DH)
        s = jnp.dot(q[:, sl], kf[:, sl].T,
                    preferred_element_type=jnp.float32) * scale
        s = s - jnp.max(s, axis=1, keepdims=True)
        p = jnp.exp(s)
        p = (p / jnp.sum(p, axis=1, keepdims=True)).astype(bf16)
        parts.append(jnp.dot(p, vf[:, sl], preferred_element_type=jnp.float32))
    ao = jnp.concatenate(parts, axis=1)

    attn_out = jnp.dot(ao, Wo_ref[...].T, preferred_element_type=jnp.float32) + bo_ref[...]
    glob_ref[0] = (jnp.dot(attn_out, Wm_ref[...].T,
                           preferred_element_type=jnp.float32) + bm_ref[...])


def _run_attn(q, k, v, Wo, bo, Wm, bm):
    full = lambda r, c: pl.BlockSpec((r, c), lambda b, nb: (0, 0))
    return pl.pallas_call(
        _attn_body,
        grid=(B, NB),
        in_specs=[
            pl.BlockSpec((1, BN, C), lambda b, nb: (b, nb, 0)),
            pl.BlockSpec((1, N, C), lambda b, nb: (b, 0, 0)),
            pl.BlockSpec((1, N, C), lambda b, nb: (b, 0, 0)),
            full(C, C), full(1, C),
            full(C, C), full(1, C),
        ],
        out_specs=pl.BlockSpec((1, BN, C), lambda b, nb: (b, nb, 0)),
        out_shape=jax.ShapeDtypeStruct((B, N, C), jnp.float32),
    )(q, k, v, Wo, bo, Wm, bm)


def _tail_body(glob_ref, loc_ref, f_ref, Wk2_ref, bk2_ref, Wc_ref, bc_ref,
               g_ref, be_ref, out_ref):
    glob = glob_ref[0]
    loc = jnp.maximum(loc_ref[...].astype(jnp.float32), 0.0)
    loc = jnp.dot(loc, Wk2_ref[...].T, preferred_element_type=jnp.float32) + bk2_ref[...]

    Wc = Wc_ref[...]
    fused = (jnp.dot(glob, Wc[:, :C].T, preferred_element_type=jnp.float32)
             + jnp.dot(loc, Wc[:, C:].T, preferred_element_type=jnp.float32)
             + bc_ref[...])
    fused = jnp.maximum(fused, 0.0)

    x = fused + f_ref[0]
    mu = jnp.mean(x, axis=1, keepdims=True)
    var = jnp.mean((x - mu) * (x - mu), axis=1, keepdims=True)
    out_ref[0] = (x - mu) / jnp.sqrt(var + 1e-5) * g_ref[...] + be_ref[...]


def _run_tail(glob, local_flat, features, Wk2, bk2, Wc, bc, gamma, beta):
    full = lambda r, c: pl.BlockSpec((r, c), lambda b, nb: (0, 0))
    return pl.pallas_call(
        _tail_body,
        grid=(B, NB),
        in_specs=[
            pl.BlockSpec((1, BN, C), lambda b, nb: (b, nb, 0)),
            pl.BlockSpec((BN, C), lambda b, nb: (b * NB + nb, 0)),
            pl.BlockSpec((1, BN, C), lambda b, nb: (b, nb, 0)),
            full(C, C), full(1, C),
            full(C, 2 * C), full(1, C),
            full(1, C), full(1, C),
        ],
        out_specs=pl.BlockSpec((1, BN, C), lambda b, nb: (b, nb, 0)),
        out_shape=jax.ShapeDtypeStruct((B, N, C), jnp.float32),
    )(glob, local_flat, features, Wk2, bk2, Wc, bc, gamma, beta)


def kernel(xyz, features, Wi, bi, Wo, bo, Wm, bm, Wk1, bk1, Wk2, bk2, Wc, bc,
           gamma, beta):
    f32 = jnp.float32
    xyzp = jnp.pad(xyz, ((0, 0), (0, 0), (0, 125))).astype(f32)
    xyzt = jnp.pad(jnp.transpose(xyz, (0, 2, 1)),
                   ((0, 0), (0, 125), (0, 0))).astype(f32)
    r2 = lambda t: t.reshape(1, -1)

    h_flat = _run_h(features, Wk1, r2(bk1))

    idx_a = _run_knn(xyzp, xyzt, 0, B // 2)
    local_a = _run_sc_gather_max(h_flat, idx_a)
    idx_b = _run_knn(xyzp, xyzt, B // 2, B // 2)
    local_b = _run_sc_gather_max(h_flat, idx_b)
    local_flat = jnp.concatenate([local_a, local_b], axis=0)

    q, k, v = _run_qkv(features, Wi, r2(bi))
    glob = _run_attn(q, k, v, Wo, r2(bo), Wm, r2(bm))

    return _run_tail(glob, local_flat, features, Wk2, r2(bk2), Wc, r2(bc),
                     r2(gamma), r2(beta))


# speedup vs baseline: 1.4632x; 1.0216x over previous
import functools
import math

import jax
import jax.numpy as jnp
from jax import lax
from jax.experimental import pallas as pl
from jax.experimental.pallas import tpu as pltpu
from jax.experimental.pallas import tpu_sc as plsc

B, N, C, K, H = 4, 2048, 512, 16, 4
DH = C // H
NB = 8
BN = N // NB
NW = 32
PPW = (B * N) // NW


def _h_body(f_ref, Wk1_ref, bk1_ref, h_ref):
    h_ref[...] = (jnp.dot(f_ref[0], Wk1_ref[...].T,
                          preferred_element_type=jnp.float32) + bk1_ref[...])


def _run_h(features, Wk1, bk1):
    return pl.pallas_call(
        _h_body,
        grid=(B, NB),
        in_specs=[
            pl.BlockSpec((1, BN, C), lambda b, nb: (b, nb, 0)),
            pl.BlockSpec((C, C), lambda b, nb: (0, 0)),
            pl.BlockSpec((1, C), lambda b, nb: (0, 0)),
        ],
        out_specs=pl.BlockSpec((BN, C), lambda b, nb: (b * NB + nb, 0)),
        out_shape=jax.ShapeDtypeStruct((B * N, C), jnp.float32),
    )(features, Wk1, bk1)


def _knn_body(b0, xyzp_ref, xyzt_ref, idx_ref):
    b = pl.program_id(0) + b0
    xb = xyzp_ref[0]
    xf = xyzt_ref[0]
    xc0, xc1, xc2 = xb[:, 0:1], xb[:, 1:2], xb[:, 2:3]
    xr0, xr1, xr2 = xf[0:1, :], xf[1:2, :], xf[2:3, :]
    dot = jnp.dot(xb, xf, preferred_element_type=jnp.float32)
    sqb = xc0 * xc0 + xc1 * xc1 + xc2 * xc2
    sqf = xr0 * xr0 + xr1 * xr1 + xr2 * xr2
    d2 = sqb + sqf - 2.0 * dot

    iota = lax.broadcasted_iota(jnp.int32, (BN, N), 1)
    col = lax.broadcasted_iota(jnp.int32, (BN, 128), 1)
    acc = jnp.zeros((BN, 128), jnp.int32)
    work = d2
    for kk in range(K):
        m = jnp.min(work, axis=1, keepdims=True)
        am = jnp.min(jnp.where(work == m, iota, N), axis=1, keepdims=True)
        acc = jnp.where(col == kk, am + b * N, acc)
        work = jnp.where(iota == am, jnp.float32(jnp.inf), work)
    idx_ref[...] = acc


def _run_knn(xyzp, xyzt, b0, nbat):
    return pl.pallas_call(
        functools.partial(_knn_body, b0),
        grid=(nbat, NB),
        in_specs=[
            pl.BlockSpec((1, BN, 128), lambda b, nb: (b0 + b, nb, 0)),
            pl.BlockSpec((1, 128, N), lambda b, nb: (b0 + b, 0, 0)),
        ],
        out_specs=pl.BlockSpec((BN, 128), lambda b, nb: (b * NB + nb, 0)),
        out_shape=jax.ShapeDtypeStruct((nbat * N, 128), jnp.int32),
    )(xyzp, xyzt)


def _qkv_body(f_ref, Wi_ref, bi_ref, q_ref, k_ref, v_ref):
    qkv = jnp.dot(f_ref[0], Wi_ref[...].T, preferred_element_type=jnp.float32)
    qkv = (qkv + bi_ref[...]).astype(jnp.bfloat16)
    q_ref[0] = qkv[:, :C]
    k_ref[0] = qkv[:, C:2 * C]
    v_ref[0] = qkv[:, 2 * C:]


def _run_qkv(features, Wi, bi):
    f32 = jnp.float32
    return pl.pallas_call(
        _qkv_body,
        grid=(B, NB),
        in_specs=[
            pl.BlockSpec((1, BN, C), lambda b, nb: (b, nb, 0)),
            pl.BlockSpec((3 * C, C), lambda b, nb: (0, 0)),
            pl.BlockSpec((1, 3 * C), lambda b, nb: (0, 0)),
        ],
        out_specs=[
            pl.BlockSpec((1, BN, C), lambda b, nb: (b, nb, 0)),
            pl.BlockSpec((1, BN, C), lambda b, nb: (b, nb, 0)),
            pl.BlockSpec((1, BN, C), lambda b, nb: (b, nb, 0)),
        ],
        out_shape=[
            jax.ShapeDtypeStruct((B, N, C), jnp.bfloat16),
            jax.ShapeDtypeStruct((B, N, C), jnp.bfloat16),
            jax.ShapeDtypeStruct((B, N, C), jnp.bfloat16),
        ],
    )(features, Wi, bi)


_OB = 64


def _sc_body(ppw, h_hbm, idx_hbm, out_hbm, idxv, rows0, rows1, outs,
             sem0, sem1):
    c = lax.axis_index("c")
    s = lax.axis_index("s")
    wid = s * 2 + c
    base = wid * ppw
    pltpu.sync_copy(idx_hbm.at[pl.ds(base, ppw)], idxv)

    def fire(p, rbuf, sem):
        pltpu.async_copy(h_hbm.at[idxv.at[p, pl.ds(0, K)]], rbuf, sem)

    def wait(p, rbuf, sem):
        pltpu.make_async_copy(h_hbm.at[idxv.at[p, pl.ds(0, K)]], rbuf, sem).wait()

    def reduce_into(rbuf, orow):
        for cc in range(C // 16):
            acc = rbuf[0, pl.ds(cc * 16, 16)]
            for r in range(1, K):
                acc = jnp.maximum(acc, rbuf[r, pl.ds(cc * 16, 16)])
            outs[orow, pl.ds(cc * 16, 16)] = acc

    fire(0, rows0, sem0)
    for ob in range(ppw // _OB):
        def pair(i, carry):
            p0 = ob * _OB + 2 * i
            p1 = p0 + 1
            pn = jnp.minimum(p1 + 1, ppw - 1)
            fire(p1, rows1, sem1)
            wait(p0, rows0, sem0)
            reduce_into(rows0, 2 * i)
            fire(pn, rows0, sem0)
            wait(p1, rows1, sem1)
            reduce_into(rows1, 2 * i + 1)
            return carry

        lax.fori_loop(0, _OB // 2, pair, 0)
        pltpu.sync_copy(outs, out_hbm.at[pl.ds(base + ob * _OB, _OB)])
    wait(ppw - 1, rows0, sem0)


def _run_sc_gather_max(h_flat, idx_flat):
    tp = idx_flat.shape[0]
    ppw = tp // NW
    mesh = plsc.VectorSubcoreMesh(core_axis_name="c", subcore_axis_name="s")
    fn = functools.partial(
        pl.kernel,
        mesh=mesh,
        out_type=jax.ShapeDtypeStruct((tp, C), jnp.float32),
        scratch_types=[
            pltpu.VMEM((ppw, 128), jnp.int32),
            pltpu.VMEM((K, C), jnp.float32),
            pltpu.VMEM((K, C), jnp.float32),
            pltpu.VMEM((_OB, C), jnp.float32),
            pltpu.SemaphoreType.DMA,
            pltpu.SemaphoreType.DMA,
        ],
    )(functools.partial(_sc_body, ppw))
    return fn(h_flat, idx_flat)


def _attn_body(q_ref, k_ref, v_ref, Wo_ref, bo_ref, Wm_ref, bm_ref, glob_ref):
    bf16 = jnp.bfloat16
    q = q_ref[0]
    kf = k_ref[0]
    vf = v_ref[0]
    scale = 1.0 / math.sqrt(DH)
    parts = []
    for hh in range(H):
        sl = slice(hh * DH, (hh + 1) * DH)
        s = jnp.dot(q[:, sl], kf[:, sl].T,
                    preferred_element_type=jnp.float32) * scale
        s = s - jnp.max(s, axis=1, keepdims=True)
        p = jnp.exp(s)
        inv = 1.0 / jnp.sum(p, axis=1, keepdims=True)
        p = (p * inv).astype(bf16)
        parts.append(jnp.dot(p, vf[:, sl], preferred_element_type=jnp.float32))
    ao = jnp.concatenate(parts, axis=1)

    attn_out = jnp.dot(ao, Wo_ref[...].T, preferred_element_type=jnp.float32) + bo_ref[...]
    glob_ref[0] = (jnp.dot(attn_out, Wm_ref[...].T,
                           preferred_element_type=jnp.float32) + bm_ref[...])


def _run_attn(q, k, v, Wo, bo, Wm, bm):
    full = lambda r, c: pl.BlockSpec((r, c), lambda b, nb: (0, 0))
    return pl.pallas_call(
        _attn_body,
        grid=(B, NB),
        in_specs=[
            pl.BlockSpec((1, BN, C), lambda b, nb: (b, nb, 0)),
            pl.BlockSpec((1, N, C), lambda b, nb: (b, 0, 0)),
            pl.BlockSpec((1, N, C), lambda b, nb: (b, 0, 0)),
            full(C, C), full(1, C),
            full(C, C), full(1, C),
        ],
        out_specs=pl.BlockSpec((1, BN, C), lambda b, nb: (b, nb, 0)),
        out_shape=jax.ShapeDtypeStruct((B, N, C), jnp.float32),
    )(q, k, v, Wo, bo, Wm, bm)


def _tail_body(glob_ref, loc_ref, f_ref, Wk2_ref, bk2_ref, Wc_ref, bc_ref,
               g_ref, be_ref, out_ref):
    glob = glob_ref[0]
    loc = jnp.maximum(loc_ref[...].astype(jnp.float32), 0.0)
    loc = jnp.dot(loc, Wk2_ref[...].T, preferred_element_type=jnp.float32) + bk2_ref[...]

    Wc = Wc_ref[...]
    fused = (jnp.dot(glob, Wc[:, :C].T, preferred_element_type=jnp.float32)
             + jnp.dot(loc, Wc[:, C:].T, preferred_element_type=jnp.float32)
             + bc_ref[...])
    fused = jnp.maximum(fused, 0.0)

    x = fused + f_ref[0]
    mu = jnp.mean(x, axis=1, keepdims=True)
    var = jnp.mean((x - mu) * (x - mu), axis=1, keepdims=True)
    out_ref[0] = (x - mu) / jnp.sqrt(var + 1e-5) * g_ref[...] + be_ref[...]


def _run_tail(glob, local_flat, features, Wk2, bk2, Wc, bc, gamma, beta):
    full = lambda r, c: pl.BlockSpec((r, c), lambda b, nb: (0, 0))
    return pl.pallas_call(
        _tail_body,
        grid=(B, NB),
        in_specs=[
            pl.BlockSpec((1, BN, C), lambda b, nb: (b, nb, 0)),
            pl.BlockSpec((BN, C), lambda b, nb: (b * NB + nb, 0)),
            pl.BlockSpec((1, BN, C), lambda b, nb: (b, nb, 0)),
            full(C, C), full(1, C),
            full(C, 2 * C), full(1, C),
            full(1, C), full(1, C),
        ],
        out_specs=pl.BlockSpec((1, BN, C), lambda b, nb: (b, nb, 0)),
        out_shape=jax.ShapeDtypeStruct((B, N, C), jnp.float32),
    )(glob, local_flat, features, Wk2, bk2, Wc, bc, gamma, beta)


def kernel(xyz, features, Wi, bi, Wo, bo, Wm, bm, Wk1, bk1, Wk2, bk2, Wc, bc,
           gamma, beta):
    f32 = jnp.float32
    xyzp = jnp.pad(xyz, ((0, 0), (0, 0), (0, 125))).astype(f32)
    xyzt = jnp.pad(jnp.transpose(xyz, (0, 2, 1)),
                   ((0, 0), (0, 125), (0, 0))).astype(f32)
    r2 = lambda t: t.reshape(1, -1)

    h_flat = _run_h(features, Wk1, r2(bk1))

    idx_a = _run_knn(xyzp, xyzt, 0, B // 2)
    local_a = _run_sc_gather_max(h_flat, idx_a)
    idx_b = _run_knn(xyzp, xyzt, B // 2, B // 2)
    local_b = _run_sc_gather_max(h_flat, idx_b)
    local_flat = jnp.concatenate([local_a, local_b], axis=0)

    q, k, v = _run_qkv(features, Wi, r2(bi))
    glob = _run_attn(q, k, v, Wo, r2(bo), Wm, r2(bm))

    return _run_tail(glob, local_flat, features, Wk2, r2(bk2), Wc, r2(bc),
                     r2(gamma), r2(beta))
